# baseline (speedup 1.0000x reference)
import functools

import numpy as np

import jax
import jax.numpy as jnp
from jax import lax
from jax.experimental import pallas as pl
from jax.experimental.pallas import tpu as pltpu
from jax.experimental.pallas import tpu_sc as plsc

N = 10000
E = 320000
D = 128
BN_EPS = 1e-5

NC = 2
NS = 16
NW = NC * NS
EPW = E // NW
C = 40
NCHUNK = EPW // C
NRCH = N // C
ZT = -(-NRCH // NS)
NIB = 6

CP = C // 2


@functools.cache
def _get_sc_agg():
    mesh = plsc.VectorSubcoreMesh(
        core_axis_name="c", subcore_axis_name="s", num_cores=NC, num_subcores=NS)

    @functools.partial(
        pl.kernel,
        out_type=jax.ShapeDtypeStruct((NC, N, D), jnp.float32),
        mesh=mesh,
        scratch_types=[
            pltpu.VMEM((NIB, C), jnp.int32),
            pltpu.VMEM((NIB, C), jnp.int32),
            pltpu.VMEM((C, D), jnp.float32),
            pltpu.VMEM((C, D), jnp.float32),
            pltpu.VMEM((C, D), jnp.float32),
            pltpu.VMEM((C, D), jnp.float32),
            pltpu.VMEM((C, D), jnp.float32),
            pltpu.VMEM((C, D), jnp.float32),
            pltpu.VMEM_SHARED((N, D), jnp.float32),
            pltpu.SemaphoreType.DMA,
            pltpu.SemaphoreType.DMA,
            pltpu.SemaphoreType.DMA,
            pltpu.SemaphoreType.DMA,
            pltpu.SemaphoreType.DMA,
            pltpu.SemaphoreType.DMA,
        ],
    )
    def _sc_agg(x_hbm, e_hbm, src_hbm, dst_hbm, out_hbm,
                isrc_v, idst_v, xg0, xg1, ev0, ev1, mb0, mb1, acc_sh,
                lsem0, lsem1, ssem0, ssem1, isem0, isem1):
        cid = lax.axis_index("c")
        sid = lax.axis_index("s")
        wid = cid * NS + sid
        xg = (xg0, xg1)
        ev = (ev0, ev1)
        mb = (mb0, mb1)
        lsem = (lsem0, lsem1)
        ssem = (ssem0, ssem1)
        isem = (isem0, isem1)

        def zero_row(i, carry):
            for j in range(D // 16):
                xg0[i, pl.ds(j * 16, 16)] = jnp.zeros((16,), jnp.float32)
            return carry

        lax.fori_loop(0, C, zero_row, 0)
        for t in range(ZT):
            rchunk = sid + NS * t

            @pl.when(rchunk < NRCH)
            def _():
                pltpu.sync_copy(xg0, acc_sh.at[pl.ds(rchunk * C, C)])
        plsc.subcore_barrier()

        def islot(k):
            return lax.rem(k, NIB)

        def issue_idx(k, p):
            base = wid * EPW + k * C
            pltpu.async_copy(src_hbm.at[pl.ds(base, C)],
                             isrc_v.at[islot(k)], isem[p])
            pltpu.async_copy(dst_hbm.at[pl.ds(base, C)],
                             idst_v.at[islot(k)], isem[p])

        def wait_idx(k, p):
            base = wid * EPW + k * C
            pltpu.make_async_copy(src_hbm.at[pl.ds(base, C)],
                                  isrc_v.at[islot(k)], isem[p]).wait()
            pltpu.make_async_copy(dst_hbm.at[pl.ds(base, C)],
                                  idst_v.at[islot(k)], isem[p]).wait()

        def issue_load(k, b):
            pltpu.async_copy(x_hbm.at[isrc_v.at[islot(k)]], xg[b], lsem[b])
            pltpu.async_copy(e_hbm.at[pl.ds(wid * EPW + k * C, C)],
                             ev[b], lsem[b])

        def wait_load(k, b):
            pltpu.make_async_copy(x_hbm.at[isrc_v.at[islot(k)]], xg[b],
                                  lsem[b]).wait()
            pltpu.make_async_copy(e_hbm.at[pl.ds(wid * EPW + k * C, C)],
                                  ev[b], lsem[b]).wait()

        def compute(b):
            def row_body(i, rcarry):
                for j in range(D // 16):
                    sl = pl.ds(j * 16, 16)
                    mb[b][i, sl] = jnp.maximum(
                        xg[b][i, sl] + ev[b][i, sl], 0.0)
                return rcarry

            lax.fori_loop(0, C, row_body, 0)

        def issue_scatter(k, b):
            pltpu.async_copy(mb[b], acc_sh.at[idst_v.at[islot(k)]], ssem[b],
                             add=True)

        def wait_scatter(k, b):
            pltpu.make_async_copy(mb[b], acc_sh.at[idst_v.at[islot(k)]],
                                  ssem[b]).wait()

        def stage(k, b, first):
            wait_load(k, b)
            if not first:
                wait_scatter(k - 2, b)
            compute(b)
            issue_scatter(k, b)

            @pl.when(k + 2 < NCHUNK)
            def _():
                wait_idx(k + 2, b)
                issue_load(k + 2, b)

            @pl.when(k + 3 < NCHUNK)
            def _():
                issue_idx(k + 3, 1 - b)

        issue_idx(0, 0)
        issue_idx(1, 1)
        wait_idx(0, 0)
        issue_load(0, 0)
        issue_idx(2, 0)
        wait_idx(1, 1)
        issue_load(1, 1)

        def pair_body(g, carry):
            stage(2 * g, 0, False)
            stage(2 * g + 1, 1, False)
            return carry

        stage(0, 0, True)
        stage(1, 1, True)
        lax.fori_loop(1, NCHUNK // 2, pair_body, 0)
        wait_scatter(NCHUNK - 2, 0)
        wait_scatter(NCHUNK - 1, 1)
        plsc.subcore_barrier()

        for t in range(ZT):
            rchunk = sid + NS * t

            @pl.when(rchunk < NRCH)
            def _():
                pltpu.sync_copy(acc_sh.at[pl.ds(rchunk * C, C)],
                                out_hbm.at[cid, pl.ds(rchunk * C, C)])

    return _sc_agg


@functools.cache
def _get_sc_agg_pk():
    mesh = plsc.VectorSubcoreMesh(
        core_axis_name="c", subcore_axis_name="s", num_cores=NC, num_subcores=NS)

    @functools.partial(
        pl.kernel,
        out_type=jax.ShapeDtypeStruct((NC, N, D), jnp.float32),
        mesh=mesh,
        scratch_types=[
            pltpu.VMEM((NIB, C), jnp.int32),
            pltpu.VMEM((NIB, C), jnp.int32),
            pltpu.VMEM((C, D), jnp.float32),
            pltpu.VMEM((C, D), jnp.float32),
            pltpu.VMEM((CP, D), jnp.int32),
            pltpu.VMEM((CP, D), jnp.int32),
            pltpu.VMEM((C, D), jnp.float32),
            pltpu.VMEM((C, D), jnp.float32),
            pltpu.VMEM_SHARED((N, D), jnp.float32),
            pltpu.SemaphoreType.DMA,
            pltpu.SemaphoreType.DMA,
            pltpu.SemaphoreType.DMA,
            pltpu.SemaphoreType.DMA,
            pltpu.SemaphoreType.DMA,
            pltpu.SemaphoreType.DMA,
        ],
    )
    def _sc_agg_pk(x_hbm, e_hbm, src_hbm, dst_hbm, out_hbm,
                   isrc_v, idst_v, xg0, xg1, ev0, ev1, mb0, mb1, acc_sh,
                   lsem0, lsem1, ssem0, ssem1, isem0, isem1):
        cid = lax.axis_index("c")
        sid = lax.axis_index("s")
        wid = cid * NS + sid
        xg = (xg0, xg1)
        ev = (ev0, ev1)
        mb = (mb0, mb1)
        lsem = (lsem0, lsem1)
        ssem = (ssem0, ssem1)
        isem = (isem0, isem1)

        def zero_row(i, carry):
            for j in range(D // 16):
                xg0[i, pl.ds(j * 16, 16)] = jnp.zeros((16,), jnp.float32)
            return carry

        lax.fori_loop(0, C, zero_row, 0)
        for t in range(ZT):
            rchunk = sid + NS * t

            @pl.when(rchunk < NRCH)
            def _():
                pltpu.sync_copy(xg0, acc_sh.at[pl.ds(rchunk * C, C)])
        plsc.subcore_barrier()

        def islot(k):
            return lax.rem(k, NIB)

        def issue_idx(k, p):
            base = wid * EPW + k * C
            pltpu.async_copy(src_hbm.at[pl.ds(base, C)],
                             isrc_v.at[islot(k)], isem[p])
            pltpu.async_copy(dst_hbm.at[pl.ds(base, C)],
                             idst_v.at[islot(k)], isem[p])

        def wait_idx(k, p):
            base = wid * EPW + k * C
            pltpu.make_async_copy(src_hbm.at[pl.ds(base, C)],
                                  isrc_v.at[islot(k)], isem[p]).wait()
            pltpu.make_async_copy(dst_hbm.at[pl.ds(base, C)],
                                  idst_v.at[islot(k)], isem[p]).wait()

        def issue_load(k, b):
            pltpu.async_copy(x_hbm.at[isrc_v.at[islot(k)]], xg[b], lsem[b])
            pltpu.async_copy(e_hbm.at[wid * NCHUNK + k], ev[b], lsem[b])

        def wait_load(k, b):
            pltpu.make_async_copy(x_hbm.at[isrc_v.at[islot(k)]], xg[b],
                                  lsem[b]).wait()
            pltpu.make_async_copy(e_hbm.at[wid * NCHUNK + k], ev[b],
                                  lsem[b]).wait()

        def compute(b):
            def row_body(r, rcarry):
                for j in range(D // 16):
                    sl = pl.ds(j * 16, 16)
                    w = ev[b][r, sl]
                    lo = jax.lax.bitcast_convert_type(w << 16, jnp.float32)
                    hi = jax.lax.bitcast_convert_type(
                        w & jnp.int32(-65536), jnp.float32)
                    mb[b][2 * r, sl] = jnp.maximum(
                        xg[b][2 * r, sl] + lo, 0.0)
                    mb[b][2 * r + 1, sl] = jnp.maximum(
                        xg[b][2 * r + 1, sl] + hi, 0.0)
                return rcarry

            lax.fori_loop(0, CP, row_body, 0)

        def issue_scatter(k, b):
            pltpu.async_copy(mb[b], acc_sh.at[idst_v.at[islot(k)]], ssem[b],
                             add=True)

        def wait_scatter(k, b):
            pltpu.make_async_copy(mb[b], acc_sh.at[idst_v.at[islot(k)]],
                                  ssem[b]).wait()

        def stage(k, b, first):
            wait_load(k, b)
            if not first:
                wait_scatter(k - 2, b)
            compute(b)
            issue_scatter(k, b)

            @pl.when(k + 2 < NCHUNK)
            def _():
                wait_idx(k + 2, b)
                issue_load(k + 2, b)

            @pl.when(k + 3 < NCHUNK)
            def _():
                issue_idx(k + 3, 1 - b)

        issue_idx(0, 0)
        issue_idx(1, 1)
        wait_idx(0, 0)
        issue_load(0, 0)
        issue_idx(2, 0)
        wait_idx(1, 1)
        issue_load(1, 1)

        def pair_body(g, carry):
            stage(2 * g, 0, False)
            stage(2 * g + 1, 1, False)
            return carry

        stage(0, 0, True)
        stage(1, 1, True)
        lax.fori_loop(1, NCHUNK // 2, pair_body, 0)
        wait_scatter(NCHUNK - 2, 0)
        wait_scatter(NCHUNK - 1, 1)
        plsc.subcore_barrier()

        for t in range(ZT):
            rchunk = sid + NS * t

            @pl.when(rchunk < NRCH)
            def _():
                pltpu.sync_copy(acc_sh.at[pl.ds(rchunk * C, C)],
                                out_hbm.at[cid, pl.ds(rchunk * C, C)])

    return _sc_agg_pk


_EBLK = 2000


def _pack_pairs(y):
    yb = jax.lax.bitcast_convert_type(y.astype(jnp.bfloat16), jnp.uint16)
    yb = yb.reshape(-1, 2, D).astype(jnp.uint32)
    packed = yb[:, 0, :] | (yb[:, 1, :] << 16)
    return jax.lax.bitcast_convert_type(packed, jnp.int32)


def _edge_mlp_body(e_ref, w0_ref, b0_ref, w1_ref, b1_ref, y1_ref, y2_ref):
    y1 = jnp.maximum(
        jax.lax.dot_general(e_ref[...], w0_ref[...], (((1,), (0,)), ((), ())),
                            preferred_element_type=jnp.float32) + b0_ref[...], 0.0)
    y1_ref[...] = _pack_pairs(y1)
    y2_ref[...] = _pack_pairs(jnp.maximum(
        jax.lax.dot_general(y1, w1_ref[...], (((1,), (0,)), ((), ())),
                            preferred_element_type=jnp.float32) + b1_ref[...], 0.0))


def _edge_mlp(e, w0t, b0, w1t, b1):
    return pl.pallas_call(
        _edge_mlp_body,
        grid=(E // _EBLK,),
        in_specs=[
            pl.BlockSpec((_EBLK, D), lambda i: (i, 0)),
            pl.BlockSpec((D, D), lambda i: (0, 0)),
            pl.BlockSpec((1, D), lambda i: (0, 0)),
            pl.BlockSpec((D, D), lambda i: (0, 0)),
            pl.BlockSpec((1, D), lambda i: (0, 0)),
        ],
        out_specs=[
            pl.BlockSpec((_EBLK // 2, D), lambda i: (i, 0)),
            pl.BlockSpec((_EBLK // 2, D), lambda i: (i, 0)),
        ],
        out_shape=[
            jax.ShapeDtypeStruct((E // 2, D), jnp.int32),
            jax.ShapeDtypeStruct((E // 2, D), jnp.int32),
        ],
    )(e, w0t, b0.reshape(1, D), w1t, b1.reshape(1, D))


def _node_update_body(x_ref, p_ref, w_ref, b_ref, g_ref, be_ref, o_ref):
    h = x_ref[...] + p_ref[0] + p_ref[1]
    y = jnp.maximum(
        jax.lax.dot_general(h, w_ref[...], (((1,), (0,)), ((), ())),
                            preferred_element_type=jnp.float32) + b_ref[...], 0.0)
    mean = jnp.mean(y, axis=0, keepdims=True)
    var = jnp.mean((y - mean) ** 2, axis=0, keepdims=True)
    o_ref[...] = (y - mean) * lax.rsqrt(var + BN_EPS) * g_ref[...] + be_ref[...]


def _node_update(x, p, wt, b, g, be):
    return pl.pallas_call(
        _node_update_body,
        out_shape=jax.ShapeDtypeStruct((N, D), jnp.float32),
    )(x, p, wt, b.reshape(1, D), g.reshape(1, D), be.reshape(1, D))


def _node_final_body(x_ref, p_ref, w_ref, b_ref, init_ref, o_ref):
    h = x_ref[...] + p_ref[0] + p_ref[1]
    y = jnp.maximum(
        jax.lax.dot_general(h, w_ref[...], (((1,), (0,)), ((), ())),
                            preferred_element_type=jnp.float32) + b_ref[...], 0.0)
    o_ref[...] = y + init_ref[...]


def _node_final(x, p, wt, b, init):
    return pl.pallas_call(
        _node_final_body,
        out_shape=jax.ShapeDtypeStruct((N, D), jnp.float32),
    )(x, p, wt, b.reshape(1, D), init)


def kernel(node_feat, edge_feat, We_w, We_b, Wa_w, Wa_b, gamma, beta, edge_index):
    src = edge_index[0]
    dst = edge_index[1]

    e1p, e2p = _edge_mlp(edge_feat, We_w[0].T, We_b[0], We_w[1].T, We_b[1])
    e1p = e1p.reshape(NW * NCHUNK, CP, D)
    e2p = e2p.reshape(NW * NCHUNK, CP, D)

    p0 = _get_sc_agg()(node_feat, edge_feat, src, dst)
    x1 = _node_update(node_feat, p0, Wa_w[0].T, Wa_b[0], gamma[0], beta[0])
    sc_agg_pk = _get_sc_agg_pk()
    p1 = sc_agg_pk(x1, e1p, src, dst)
    x2 = _node_update(x1, p1, Wa_w[1].T, Wa_b[1], gamma[1], beta[1])
    p2 = sc_agg_pk(x2, e2p, src, dst)
    return _node_final(x2, p2, Wa_w[1].T, Wa_b[1], node_feat)


# speedup vs baseline: 1.4949x; 1.4949x over previous
import functools

import numpy as np

import jax
import jax.numpy as jnp
from jax import lax
from jax.experimental import pallas as pl
from jax.experimental.pallas import tpu as pltpu
from jax.experimental.pallas import tpu_sc as plsc

N = 10000
E = 320000
D = 128
BN_EPS = 1e-5

NC = 2
NS = 16
NW = NC * NS
EPW = E // NW
C = 40
NCHUNK = EPW // C
NRCH = N // C
ZT = -(-NRCH // NS)
NIB = 6

CP = C // 2


@functools.cache
def _get_sc_agg():
    mesh = plsc.VectorSubcoreMesh(
        core_axis_name="c", subcore_axis_name="s", num_cores=NC, num_subcores=NS)

    @functools.partial(
        pl.kernel,
        out_type=jax.ShapeDtypeStruct((NC, N, D), jnp.float32),
        mesh=mesh,
        scratch_types=[
            pltpu.VMEM((NIB, C), jnp.int32),
            pltpu.VMEM((NIB, C), jnp.int32),
            pltpu.VMEM((C, D), jnp.float32),
            pltpu.VMEM((C, D), jnp.float32),
            pltpu.VMEM((C, D), jnp.float32),
            pltpu.VMEM((C, D), jnp.float32),
            pltpu.VMEM((C, D), jnp.float32),
            pltpu.VMEM((C, D), jnp.float32),
            pltpu.VMEM_SHARED((N, D), jnp.float32),
            pltpu.SemaphoreType.DMA,
            pltpu.SemaphoreType.DMA,
            pltpu.SemaphoreType.DMA,
            pltpu.SemaphoreType.DMA,
            pltpu.SemaphoreType.DMA,
            pltpu.SemaphoreType.DMA,
        ],
    )
    def _sc_agg(x_hbm, e_hbm, src_hbm, dst_hbm, out_hbm,
                isrc_v, idst_v, xg0, xg1, ev0, ev1, mb0, mb1, acc_sh,
                lsem0, lsem1, ssem0, ssem1, isem0, isem1):
        cid = lax.axis_index("c")
        sid = lax.axis_index("s")
        wid = cid * NS + sid
        xg = (xg0, xg1)
        ev = (ev0, ev1)
        mb = (mb0, mb1)
        lsem = (lsem0, lsem1)
        ssem = (ssem0, ssem1)
        isem = (isem0, isem1)

        def zero_row(i, carry):
            for j in range(D // 16):
                xg0[i, pl.ds(j * 16, 16)] = jnp.zeros((16,), jnp.float32)
            return carry

        lax.fori_loop(0, C, zero_row, 0)
        for t in range(ZT):
            rchunk = sid + NS * t

            @pl.when(rchunk < NRCH)
            def _():
                pltpu.sync_copy(xg0, acc_sh.at[pl.ds(rchunk * C, C)])
        plsc.subcore_barrier()

        def islot(k):
            return lax.rem(k, NIB)

        def issue_idx(k, p):
            base = wid * EPW + k * C
            pltpu.async_copy(src_hbm.at[pl.ds(base, C)],
                             isrc_v.at[islot(k)], isem[p])
            pltpu.async_copy(dst_hbm.at[pl.ds(base, C)],
                             idst_v.at[islot(k)], isem[p])

        def wait_idx(k, p):
            base = wid * EPW + k * C
            pltpu.make_async_copy(src_hbm.at[pl.ds(base, C)],
                                  isrc_v.at[islot(k)], isem[p]).wait()
            pltpu.make_async_copy(dst_hbm.at[pl.ds(base, C)],
                                  idst_v.at[islot(k)], isem[p]).wait()

        def issue_load(k, b):
            pltpu.async_copy(x_hbm.at[isrc_v.at[islot(k)]], xg[b], lsem[b])
            pltpu.async_copy(e_hbm.at[pl.ds(wid * EPW + k * C, C)],
                             ev[b], lsem[b])

        def wait_load(k, b):
            pltpu.make_async_copy(x_hbm.at[isrc_v.at[islot(k)]], xg[b],
                                  lsem[b]).wait()
            pltpu.make_async_copy(e_hbm.at[pl.ds(wid * EPW + k * C, C)],
                                  ev[b], lsem[b]).wait()

        def compute(b):
            def row_body(i, rcarry):
                for j in range(D // 16):
                    sl = pl.ds(j * 16, 16)
                    mb[b][i, sl] = jnp.maximum(
                        xg[b][i, sl] + ev[b][i, sl], 0.0)
                return rcarry

            lax.fori_loop(0, C, row_body, 0)

        def issue_scatter(k, b):
            pltpu.async_copy(mb[b], acc_sh.at[idst_v.at[islot(k)]], ssem[b],
                             add=True)

        def wait_scatter(k, b):
            pltpu.make_async_copy(mb[b], acc_sh.at[idst_v.at[islot(k)]],
                                  ssem[b]).wait()

        def stage(k, b, first):
            wait_load(k, b)
            if not first:
                wait_scatter(k - 2, b)
            compute(b)
            issue_scatter(k, b)

            @pl.when(k + 2 < NCHUNK)
            def _():
                wait_idx(k + 2, b)
                issue_load(k + 2, b)

            @pl.when(k + 3 < NCHUNK)
            def _():
                issue_idx(k + 3, 1 - b)

        issue_idx(0, 0)
        issue_idx(1, 1)
        wait_idx(0, 0)
        issue_load(0, 0)
        issue_idx(2, 0)
        wait_idx(1, 1)
        issue_load(1, 1)

        def pair_body(g, carry):
            stage(2 * g, 0, False)
            stage(2 * g + 1, 1, False)
            return carry

        stage(0, 0, True)
        stage(1, 1, True)
        lax.fori_loop(1, NCHUNK // 2, pair_body, 0)
        wait_scatter(NCHUNK - 2, 0)
        wait_scatter(NCHUNK - 1, 1)
        plsc.subcore_barrier()

        for t in range(ZT):
            rchunk = sid + NS * t

            @pl.when(rchunk < NRCH)
            def _():
                pltpu.sync_copy(acc_sh.at[pl.ds(rchunk * C, C)],
                                out_hbm.at[cid, pl.ds(rchunk * C, C)])

    return _sc_agg


@functools.cache
def _get_sc_agg_pk():
    mesh = plsc.VectorSubcoreMesh(
        core_axis_name="c", subcore_axis_name="s", num_cores=NC, num_subcores=NS)

    @functools.partial(
        pl.kernel,
        out_type=jax.ShapeDtypeStruct((NC, N, D), jnp.float32),
        mesh=mesh,
        scratch_types=[
            pltpu.VMEM((NIB, C), jnp.int32),
            pltpu.VMEM((NIB, C), jnp.int32),
            pltpu.VMEM((C, D), jnp.float32),
            pltpu.VMEM((C, D), jnp.float32),
            pltpu.VMEM((C, D // 2), jnp.int32),
            pltpu.VMEM((C, D // 2), jnp.int32),
            pltpu.VMEM((C, D), jnp.float32),
            pltpu.VMEM((C, D), jnp.float32),
            pltpu.VMEM_SHARED((N, D), jnp.float32),
            pltpu.SemaphoreType.DMA,
            pltpu.SemaphoreType.DMA,
            pltpu.SemaphoreType.DMA,
            pltpu.SemaphoreType.DMA,
            pltpu.SemaphoreType.DMA,
            pltpu.SemaphoreType.DMA,
        ],
    )
    def _sc_agg_pk(x_hbm, e_hbm, src_hbm, dst_hbm, out_hbm,
                   isrc_v, idst_v, xg0, xg1, ev0, ev1, mb0, mb1, acc_sh,
                   lsem0, lsem1, ssem0, ssem1, isem0, isem1):
        cid = lax.axis_index("c")
        sid = lax.axis_index("s")
        wid = cid * NS + sid
        xg = (xg0, xg1)
        ev = (ev0, ev1)
        mb = (mb0, mb1)
        lsem = (lsem0, lsem1)
        ssem = (ssem0, ssem1)
        isem = (isem0, isem1)

        def zero_row(i, carry):
            for j in range(D // 16):
                xg0[i, pl.ds(j * 16, 16)] = jnp.zeros((16,), jnp.float32)
            return carry

        lax.fori_loop(0, C, zero_row, 0)
        for t in range(ZT):
            rchunk = sid + NS * t

            @pl.when(rchunk < NRCH)
            def _():
                pltpu.sync_copy(xg0, acc_sh.at[pl.ds(rchunk * C, C)])
        plsc.subcore_barrier()

        def islot(k):
            return lax.rem(k, NIB)

        def issue_idx(k, p):
            base = wid * EPW + k * C
            pltpu.async_copy(src_hbm.at[pl.ds(base, C)],
                             isrc_v.at[islot(k)], isem[p])
            pltpu.async_copy(dst_hbm.at[pl.ds(base, C)],
                             idst_v.at[islot(k)], isem[p])

        def wait_idx(k, p):
            base = wid * EPW + k * C
            pltpu.make_async_copy(src_hbm.at[pl.ds(base, C)],
                                  isrc_v.at[islot(k)], isem[p]).wait()
            pltpu.make_async_copy(dst_hbm.at[pl.ds(base, C)],
                                  idst_v.at[islot(k)], isem[p]).wait()

        def issue_load(k, b):
            pltpu.async_copy(x_hbm.at[isrc_v.at[islot(k)]], xg[b], lsem[b])
            pltpu.async_copy(e_hbm.at[pl.ds(wid * EPW + k * C, C)],
                             ev[b], lsem[b])

        def wait_load(k, b):
            pltpu.make_async_copy(x_hbm.at[isrc_v.at[islot(k)]], xg[b],
                                  lsem[b]).wait()
            pltpu.make_async_copy(e_hbm.at[pl.ds(wid * EPW + k * C, C)],
                                  ev[b], lsem[b]).wait()

        def compute(b):
            def row_body(r, rcarry):
                for j in range(D // 32):
                    sl = pl.ds(j * 16, 16)
                    sh = pl.ds(j * 16 + D // 2, 16)
                    w = ev[b][r, sl]
                    lo = jax.lax.bitcast_convert_type(w << 16, jnp.float32)
                    hi = jax.lax.bitcast_convert_type(
                        w & jnp.int32(-65536), jnp.float32)
                    mb[b][r, sl] = jnp.maximum(xg[b][r, sl] + lo, 0.0)
                    mb[b][r, sh] = jnp.maximum(xg[b][r, sh] + hi, 0.0)
                return rcarry

            lax.fori_loop(0, C, row_body, 0)

        def issue_scatter(k, b):
            pltpu.async_copy(mb[b], acc_sh.at[idst_v.at[islot(k)]], ssem[b],
                             add=True)

        def wait_scatter(k, b):
            pltpu.make_async_copy(mb[b], acc_sh.at[idst_v.at[islot(k)]],
                                  ssem[b]).wait()

        def stage(k, b, first):
            wait_load(k, b)
            if not first:
                wait_scatter(k - 2, b)
            compute(b)
            issue_scatter(k, b)

            @pl.when(k + 2 < NCHUNK)
            def _():
                wait_idx(k + 2, b)
                issue_load(k + 2, b)

            @pl.when(k + 3 < NCHUNK)
            def _():
                issue_idx(k + 3, 1 - b)

        issue_idx(0, 0)
        issue_idx(1, 1)
        wait_idx(0, 0)
        issue_load(0, 0)
        issue_idx(2, 0)
        wait_idx(1, 1)
        issue_load(1, 1)

        def pair_body(g, carry):
            stage(2 * g, 0, False)
            stage(2 * g + 1, 1, False)
            return carry

        stage(0, 0, True)
        stage(1, 1, True)
        lax.fori_loop(1, NCHUNK // 2, pair_body, 0)
        wait_scatter(NCHUNK - 2, 0)
        wait_scatter(NCHUNK - 1, 1)
        plsc.subcore_barrier()

        for t in range(ZT):
            rchunk = sid + NS * t

            @pl.when(rchunk < NRCH)
            def _():
                pltpu.sync_copy(acc_sh.at[pl.ds(rchunk * C, C)],
                                out_hbm.at[cid, pl.ds(rchunk * C, C)])

    return _sc_agg_pk


_EBLK = 2000


def _pack_cols(y):
    yb = jax.lax.bitcast_convert_type(y.astype(jnp.bfloat16), jnp.uint16)
    yb = yb.astype(jnp.uint32)
    packed = yb[:, :D // 2] | (yb[:, D // 2:] << 16)
    return jax.lax.bitcast_convert_type(packed, jnp.int32)


def _edge_mlp_body(e_ref, w0_ref, b0_ref, w1_ref, b1_ref, y1_ref, y2_ref):
    y1 = jnp.maximum(
        jax.lax.dot_general(e_ref[...], w0_ref[...], (((1,), (0,)), ((), ())),
                            preferred_element_type=jnp.float32) + b0_ref[...], 0.0)
    y1_ref[...] = _pack_cols(y1)
    y2_ref[...] = _pack_cols(jnp.maximum(
        jax.lax.dot_general(y1, w1_ref[...], (((1,), (0,)), ((), ())),
                            preferred_element_type=jnp.float32) + b1_ref[...], 0.0))


def _edge_mlp(e, w0t, b0, w1t, b1):
    return pl.pallas_call(
        _edge_mlp_body,
        grid=(E // _EBLK,),
        in_specs=[
            pl.BlockSpec((_EBLK, D), lambda i: (i, 0)),
            pl.BlockSpec((D, D), lambda i: (0, 0)),
            pl.BlockSpec((1, D), lambda i: (0, 0)),
            pl.BlockSpec((D, D), lambda i: (0, 0)),
            pl.BlockSpec((1, D), lambda i: (0, 0)),
        ],
        out_specs=[
            pl.BlockSpec((_EBLK, D // 2), lambda i: (i, 0)),
            pl.BlockSpec((_EBLK, D // 2), lambda i: (i, 0)),
        ],
        out_shape=[
            jax.ShapeDtypeStruct((E, D // 2), jnp.int32),
            jax.ShapeDtypeStruct((E, D // 2), jnp.int32),
        ],
    )(e, w0t, b0.reshape(1, D), w1t, b1.reshape(1, D))


def _node_update_body(x_ref, p_ref, w_ref, b_ref, g_ref, be_ref, o_ref):
    h = x_ref[...] + p_ref[0] + p_ref[1]
    y = jnp.maximum(
        jax.lax.dot_general(h, w_ref[...], (((1,), (0,)), ((), ())),
                            preferred_element_type=jnp.float32) + b_ref[...], 0.0)
    mean = jnp.mean(y, axis=0, keepdims=True)
    var = jnp.mean((y - mean) ** 2, axis=0, keepdims=True)
    o_ref[...] = (y - mean) * lax.rsqrt(var + BN_EPS) * g_ref[...] + be_ref[...]


def _node_update(x, p, wt, b, g, be):
    return pl.pallas_call(
        _node_update_body,
        out_shape=jax.ShapeDtypeStruct((N, D), jnp.float32),
    )(x, p, wt, b.reshape(1, D), g.reshape(1, D), be.reshape(1, D))


def _node_final_body(x_ref, p_ref, w_ref, b_ref, init_ref, o_ref):
    h = x_ref[...] + p_ref[0] + p_ref[1]
    y = jnp.maximum(
        jax.lax.dot_general(h, w_ref[...], (((1,), (0,)), ((), ())),
                            preferred_element_type=jnp.float32) + b_ref[...], 0.0)
    o_ref[...] = y + init_ref[...]


def _node_final(x, p, wt, b, init):
    return pl.pallas_call(
        _node_final_body,
        out_shape=jax.ShapeDtypeStruct((N, D), jnp.float32),
    )(x, p, wt, b.reshape(1, D), init)


def kernel(node_feat, edge_feat, We_w, We_b, Wa_w, Wa_b, gamma, beta, edge_index):
    src = edge_index[0]
    dst = edge_index[1]

    e1p, e2p = _edge_mlp(edge_feat, We_w[0].T, We_b[0], We_w[1].T, We_b[1])

    p0 = _get_sc_agg()(node_feat, edge_feat, src, dst)
    x1 = _node_update(node_feat, p0, Wa_w[0].T, Wa_b[0], gamma[0], beta[0])
    sc_agg_pk = _get_sc_agg_pk()
    p1 = sc_agg_pk(x1, e1p, src, dst)
    x2 = _node_update(x1, p1, Wa_w[1].T, Wa_b[1], gamma[1], beta[1])
    p2 = sc_agg_pk(x2, e2p, src, dst)
    return _node_final(x2, p2, Wa_w[1].T, Wa_b[1], node_feat)


# speedup vs baseline: 1.6566x; 1.1082x over previous
import functools

import numpy as np

import jax
import jax.numpy as jnp
from jax import lax
from jax.experimental import pallas as pl
from jax.experimental.pallas import tpu as pltpu
from jax.experimental.pallas import tpu_sc as plsc

N = 10000
E = 320000
D = 128
BN_EPS = 1e-5

NC = 2
NS = 16
NW = NC * NS
EPW = E // NW
C = 40
NCHUNK = EPW // C
NRCH = N // C
ZT = -(-NRCH // NS)
NIB = 6

CP = C // 2


@functools.cache
def _get_sc_agg():
    mesh = plsc.VectorSubcoreMesh(
        core_axis_name="c", subcore_axis_name="s", num_cores=NC, num_subcores=NS)

    @functools.partial(
        pl.kernel,
        out_type=jax.ShapeDtypeStruct((NC, N, D), jnp.float32),
        mesh=mesh,
        scratch_types=[
            pltpu.VMEM((NIB, C), jnp.int32),
            pltpu.VMEM((NIB, C), jnp.int32),
            pltpu.VMEM((C, D), jnp.float32),
            pltpu.VMEM((C, D), jnp.float32),
            pltpu.VMEM((C, D), jnp.float32),
            pltpu.VMEM((C, D), jnp.float32),
            pltpu.VMEM((C, D), jnp.float32),
            pltpu.VMEM((C, D), jnp.float32),
            pltpu.VMEM_SHARED((N, D), jnp.float32),
            pltpu.SemaphoreType.DMA,
            pltpu.SemaphoreType.DMA,
            pltpu.SemaphoreType.DMA,
            pltpu.SemaphoreType.DMA,
            pltpu.SemaphoreType.DMA,
            pltpu.SemaphoreType.DMA,
        ],
    )
    def _sc_agg(x_hbm, e_hbm, src_hbm, dst_hbm, out_hbm,
                isrc_v, idst_v, xg0, xg1, ev0, ev1, mb0, mb1, acc_sh,
                lsem0, lsem1, ssem0, ssem1, isem0, isem1):
        cid = lax.axis_index("c")
        sid = lax.axis_index("s")
        wid = cid * NS + sid
        xg = (xg0, xg1)
        ev = (ev0, ev1)
        mb = (mb0, mb1)
        lsem = (lsem0, lsem1)
        ssem = (ssem0, ssem1)
        isem = (isem0, isem1)

        def zero_row(i, carry):
            for j in range(D // 16):
                xg0[i, pl.ds(j * 16, 16)] = jnp.zeros((16,), jnp.float32)
            return carry

        lax.fori_loop(0, C, zero_row, 0)
        for t in range(ZT):
            rchunk = sid + NS * t

            @pl.when(rchunk < NRCH)
            def _():
                pltpu.sync_copy(xg0, acc_sh.at[pl.ds(rchunk * C, C)])
        plsc.subcore_barrier()

        def islot(k):
            return lax.rem(k, NIB)

        def issue_idx(k, p):
            base = wid * EPW + k * C
            pltpu.async_copy(src_hbm.at[pl.ds(base, C)],
                             isrc_v.at[islot(k)], isem[p])
            pltpu.async_copy(dst_hbm.at[pl.ds(base, C)],
                             idst_v.at[islot(k)], isem[p])

        def wait_idx(k, p):
            base = wid * EPW + k * C
            pltpu.make_async_copy(src_hbm.at[pl.ds(base, C)],
                                  isrc_v.at[islot(k)], isem[p]).wait()
            pltpu.make_async_copy(dst_hbm.at[pl.ds(base, C)],
                                  idst_v.at[islot(k)], isem[p]).wait()

        def issue_load(k, b):
            pltpu.async_copy(x_hbm.at[isrc_v.at[islot(k)]], xg[b], lsem[b])
            pltpu.async_copy(e_hbm.at[pl.ds(wid * EPW + k * C, C)],
                             ev[b], lsem[b])

        def wait_load(k, b):
            pltpu.make_async_copy(x_hbm.at[isrc_v.at[islot(k)]], xg[b],
                                  lsem[b]).wait()
            pltpu.make_async_copy(e_hbm.at[pl.ds(wid * EPW + k * C, C)],
                                  ev[b], lsem[b]).wait()

        def compute(b):
            def row_body(i, rcarry):
                for j in range(D // 16):
                    sl = pl.ds(j * 16, 16)
                    mb[b][i, sl] = jnp.maximum(
                        xg[b][i, sl] + ev[b][i, sl], 0.0)
                return rcarry

            lax.fori_loop(0, C, row_body, 0)

        def issue_scatter(k, b):
            pltpu.async_copy(mb[b], acc_sh.at[idst_v.at[islot(k)]], ssem[b],
                             add=True)

        def wait_scatter(k, b):
            pltpu.make_async_copy(mb[b], acc_sh.at[idst_v.at[islot(k)]],
                                  ssem[b]).wait()

        def stage(k, b, first):
            wait_load(k, b)
            if not first:
                wait_scatter(k - 2, b)
            compute(b)
            issue_scatter(k, b)

            @pl.when(k + 2 < NCHUNK)
            def _():
                wait_idx(k + 2, b)
                issue_load(k + 2, b)

            @pl.when(k + 3 < NCHUNK)
            def _():
                issue_idx(k + 3, 1 - b)

        issue_idx(0, 0)
        issue_idx(1, 1)
        wait_idx(0, 0)
        issue_load(0, 0)
        issue_idx(2, 0)
        wait_idx(1, 1)
        issue_load(1, 1)

        def pair_body(g, carry):
            stage(2 * g, 0, False)
            stage(2 * g + 1, 1, False)
            return carry

        stage(0, 0, True)
        stage(1, 1, True)
        lax.fori_loop(1, NCHUNK // 2, pair_body, 0)
        wait_scatter(NCHUNK - 2, 0)
        wait_scatter(NCHUNK - 1, 1)
        plsc.subcore_barrier()

        for t in range(ZT):
            rchunk = sid + NS * t

            @pl.when(rchunk < NRCH)
            def _():
                pltpu.sync_copy(acc_sh.at[pl.ds(rchunk * C, C)],
                                out_hbm.at[cid, pl.ds(rchunk * C, C)])

    return _sc_agg


@functools.cache
def _get_sc_agg_pk():
    mesh = plsc.VectorSubcoreMesh(
        core_axis_name="c", subcore_axis_name="s", num_cores=NC, num_subcores=NS)

    @functools.partial(
        pl.kernel,
        out_type=jax.ShapeDtypeStruct((NC, N, D), jnp.float32),
        mesh=mesh,
        scratch_types=[
            pltpu.VMEM((NIB, C), jnp.int32),
            pltpu.VMEM((NIB, C), jnp.int32),
            pltpu.VMEM((C, D), jnp.float32),
            pltpu.VMEM((C, D), jnp.float32),
            pltpu.VMEM((C, D), jnp.float32),
            pltpu.VMEM((C, D // 2), jnp.int32),
            pltpu.VMEM((C, D // 2), jnp.int32),
            pltpu.VMEM((C, D // 2), jnp.int32),
            pltpu.VMEM((C, D), jnp.float32),
            pltpu.VMEM((C, D), jnp.float32),
            pltpu.VMEM_SHARED((N, D), jnp.float32),
            pltpu.SemaphoreType.DMA,
            pltpu.SemaphoreType.DMA,
            pltpu.SemaphoreType.DMA,
            pltpu.SemaphoreType.DMA,
            pltpu.SemaphoreType.DMA,
            pltpu.SemaphoreType.DMA,
        ],
    )
    def _sc_agg_pk(x_hbm, e_hbm, src_hbm, dst_hbm, out_hbm,
                   isrc_v, idst_v, xg0, xg1, xg2, ev0, ev1, ev2, mb0, mb1,
                   acc_sh, lsem0, lsem1, lsem2, ssem0, ssem1, isem):
        cid = lax.axis_index("c")
        sid = lax.axis_index("s")
        wid = cid * NS + sid
        xg = (xg0, xg1, xg2)
        ev = (ev0, ev1, ev2)
        mb = (mb0, mb1)
        lsem = (lsem0, lsem1, lsem2)
        ssem = (ssem0, ssem1)

        def zero_row(i, carry):
            for j in range(D // 16):
                xg0[i, pl.ds(j * 16, 16)] = jnp.zeros((16,), jnp.float32)
            return carry

        lax.fori_loop(0, C, zero_row, 0)
        for t in range(ZT):
            rchunk = sid + NS * t

            @pl.when(rchunk < NRCH)
            def _():
                pltpu.sync_copy(xg0, acc_sh.at[pl.ds(rchunk * C, C)])
        plsc.subcore_barrier()

        def islot(k):
            return lax.rem(k, NIB)

        def issue_idx(k):
            base = wid * EPW + k * C
            pltpu.async_copy(src_hbm.at[pl.ds(base, C)],
                             isrc_v.at[islot(k)], isem)
            pltpu.async_copy(dst_hbm.at[pl.ds(base, C)],
                             idst_v.at[islot(k)], isem)

        def wait_idx(k):
            base = wid * EPW + k * C
            pltpu.make_async_copy(src_hbm.at[pl.ds(base, C)],
                                  isrc_v.at[islot(k)], isem).wait()
            pltpu.make_async_copy(dst_hbm.at[pl.ds(base, C)],
                                  idst_v.at[islot(k)], isem).wait()

        def issue_load(k, b):
            pltpu.async_copy(x_hbm.at[isrc_v.at[islot(k)]], xg[b], lsem[b])
            pltpu.async_copy(e_hbm.at[pl.ds(wid * EPW + k * C, C)],
                             ev[b], lsem[b])

        def wait_load(k, b):
            pltpu.make_async_copy(x_hbm.at[isrc_v.at[islot(k)]], xg[b],
                                  lsem[b]).wait()
            pltpu.make_async_copy(e_hbm.at[pl.ds(wid * EPW + k * C, C)],
                                  ev[b], lsem[b]).wait()

        def compute(b3, b2):
            def row_body(r, rcarry):
                for j in range(D // 32):
                    sl = pl.ds(j * 16, 16)
                    sh = pl.ds(j * 16 + D // 2, 16)
                    w = ev[b3][r, sl]
                    lo = jax.lax.bitcast_convert_type(w << 16, jnp.float32)
                    hi = jax.lax.bitcast_convert_type(
                        w & jnp.int32(-65536), jnp.float32)
                    mb[b2][r, sl] = jnp.maximum(xg[b3][r, sl] + lo, 0.0)
                    mb[b2][r, sh] = jnp.maximum(xg[b3][r, sh] + hi, 0.0)
                return rcarry

            lax.fori_loop(0, C, row_body, 0)

        def issue_scatter(k, b2):
            pltpu.async_copy(mb[b2], acc_sh.at[idst_v.at[islot(k)]],
                             ssem[b2], add=True)

        def wait_scatter(k, b2):
            pltpu.make_async_copy(mb[b2], acc_sh.at[idst_v.at[islot(k)]],
                                  ssem[b2]).wait()

        def stage(k, b3, b2, first):
            wait_load(k, b3)
            if not first:
                wait_scatter(k - 2, b2)
            compute(b3, b2)
            issue_scatter(k, b2)

            @pl.when(k + 3 < NCHUNK)
            def _():
                wait_idx(k + 3)
                issue_load(k + 3, b3)

            @pl.when(k + 4 < NCHUNK)
            def _():
                issue_idx(k + 4)

        issue_idx(0)
        wait_idx(0)
        issue_load(0, 0)
        issue_idx(1)
        wait_idx(1)
        issue_load(1, 1)
        issue_idx(2)
        wait_idx(2)
        issue_load(2, 2)
        issue_idx(3)

        stage(0, 0, 0, True)
        stage(1, 1, 1, True)
        stage(2, 2, 0, False)
        stage(3, 0, 1, False)

        def six_body(g, carry):
            s = 4 + 6 * g
            stage(s, 1, 0, False)
            stage(s + 1, 2, 1, False)
            stage(s + 2, 0, 0, False)
            stage(s + 3, 1, 1, False)
            stage(s + 4, 2, 0, False)
            stage(s + 5, 0, 1, False)
            return carry

        lax.fori_loop(0, (NCHUNK - 4) // 6, six_body, 0)
        wait_scatter(NCHUNK - 2, 0)
        wait_scatter(NCHUNK - 1, 1)
        plsc.subcore_barrier()

        for t in range(ZT):
            rchunk = sid + NS * t

            @pl.when(rchunk < NRCH)
            def _():
                pltpu.sync_copy(acc_sh.at[pl.ds(rchunk * C, C)],
                                out_hbm.at[cid, pl.ds(rchunk * C, C)])

    return _sc_agg_pk


_EBLK = 2000


def _pack_cols(y):
    yb = jax.lax.bitcast_convert_type(y.astype(jnp.bfloat16), jnp.uint16)
    yb = yb.astype(jnp.uint32)
    packed = yb[:, :D // 2] | (yb[:, D // 2:] << 16)
    return jax.lax.bitcast_convert_type(packed, jnp.int32)


def _edge_mlp_body(e_ref, w0_ref, b0_ref, w1_ref, b1_ref, y1_ref, y2_ref):
    y1 = jnp.maximum(
        jax.lax.dot_general(e_ref[...], w0_ref[...], (((1,), (0,)), ((), ())),
                            preferred_element_type=jnp.float32) + b0_ref[...], 0.0)
    y1_ref[...] = _pack_cols(y1)
    y2_ref[...] = _pack_cols(jnp.maximum(
        jax.lax.dot_general(y1, w1_ref[...], (((1,), (0,)), ((), ())),
                            preferred_element_type=jnp.float32) + b1_ref[...], 0.0))


def _edge_mlp(e, w0t, b0, w1t, b1):
    return pl.pallas_call(
        _edge_mlp_body,
        grid=(E // _EBLK,),
        in_specs=[
            pl.BlockSpec((_EBLK, D), lambda i: (i, 0)),
            pl.BlockSpec((D, D), lambda i: (0, 0)),
            pl.BlockSpec((1, D), lambda i: (0, 0)),
            pl.BlockSpec((D, D), lambda i: (0, 0)),
            pl.BlockSpec((1, D), lambda i: (0, 0)),
        ],
        out_specs=[
            pl.BlockSpec((_EBLK, D // 2), lambda i: (i, 0)),
            pl.BlockSpec((_EBLK, D // 2), lambda i: (i, 0)),
        ],
        out_shape=[
            jax.ShapeDtypeStruct((E, D // 2), jnp.int32),
            jax.ShapeDtypeStruct((E, D // 2), jnp.int32),
        ],
    )(e, w0t, b0.reshape(1, D), w1t, b1.reshape(1, D))


def _node_update_body(x_ref, p_ref, w_ref, b_ref, g_ref, be_ref, o_ref):
    h = x_ref[...] + p_ref[0] + p_ref[1]
    y = jnp.maximum(
        jax.lax.dot_general(h, w_ref[...], (((1,), (0,)), ((), ())),
                            preferred_element_type=jnp.float32) + b_ref[...], 0.0)
    mean = jnp.mean(y, axis=0, keepdims=True)
    var = jnp.mean((y - mean) ** 2, axis=0, keepdims=True)
    o_ref[...] = (y - mean) * lax.rsqrt(var + BN_EPS) * g_ref[...] + be_ref[...]


def _node_update(x, p, wt, b, g, be):
    return pl.pallas_call(
        _node_update_body,
        out_shape=jax.ShapeDtypeStruct((N, D), jnp.float32),
    )(x, p, wt, b.reshape(1, D), g.reshape(1, D), be.reshape(1, D))


def _node_final_body(x_ref, p_ref, w_ref, b_ref, init_ref, o_ref):
    h = x_ref[...] + p_ref[0] + p_ref[1]
    y = jnp.maximum(
        jax.lax.dot_general(h, w_ref[...], (((1,), (0,)), ((), ())),
                            preferred_element_type=jnp.float32) + b_ref[...], 0.0)
    o_ref[...] = y + init_ref[...]


def _node_final(x, p, wt, b, init):
    return pl.pallas_call(
        _node_final_body,
        out_shape=jax.ShapeDtypeStruct((N, D), jnp.float32),
    )(x, p, wt, b.reshape(1, D), init)


def kernel(node_feat, edge_feat, We_w, We_b, Wa_w, Wa_b, gamma, beta, edge_index):
    src = edge_index[0]
    dst = edge_index[1]

    e1p, e2p = _edge_mlp(edge_feat, We_w[0].T, We_b[0], We_w[1].T, We_b[1])

    p0 = _get_sc_agg()(node_feat, edge_feat, src, dst)
    x1 = _node_update(node_feat, p0, Wa_w[0].T, Wa_b[0], gamma[0], beta[0])
    sc_agg_pk = _get_sc_agg_pk()
    p1 = sc_agg_pk(x1, e1p, src, dst)
    x2 = _node_update(x1, p1, Wa_w[1].T, Wa_b[1], gamma[1], beta[1])
    p2 = sc_agg_pk(x2, e2p, src, dst)
    return _node_final(x2, p2, Wa_w[1].T, Wa_b[1], node_feat)


# speedup vs baseline: 1.6846x; 1.0170x over previous
import functools

import numpy as np

import jax
import jax.numpy as jnp
from jax import lax
from jax.experimental import pallas as pl
from jax.experimental.pallas import tpu as pltpu
from jax.experimental.pallas import tpu_sc as plsc

N = 10000
E = 320000
D = 128
BN_EPS = 1e-5

NC = 2
NS = 16
NW = NC * NS
EPW = E // NW
C = 40
NCHUNK = EPW // C
NRCH = N // C
ZT = -(-NRCH // NS)
NIB = 6

CP = C // 2


@functools.cache
def _get_sc_agg():
    mesh = plsc.VectorSubcoreMesh(
        core_axis_name="c", subcore_axis_name="s", num_cores=NC, num_subcores=NS)

    @functools.partial(
        pl.kernel,
        out_type=jax.ShapeDtypeStruct((NC, N, D), jnp.float32),
        mesh=mesh,
        scratch_types=[
            pltpu.VMEM((NIB, C), jnp.int32),
            pltpu.VMEM((NIB, C), jnp.int32),
            pltpu.VMEM((C, D), jnp.float32),
            pltpu.VMEM((C, D), jnp.float32),
            pltpu.VMEM((C, D), jnp.float32),
            pltpu.VMEM((C, D), jnp.float32),
            pltpu.VMEM((C, D), jnp.float32),
            pltpu.VMEM((C, D), jnp.float32),
            pltpu.VMEM((C, D), jnp.float32),
            pltpu.VMEM_SHARED((N, D), jnp.float32),
            pltpu.SemaphoreType.DMA,
            pltpu.SemaphoreType.DMA,
            pltpu.SemaphoreType.DMA,
            pltpu.SemaphoreType.DMA,
            pltpu.SemaphoreType.DMA,
            pltpu.SemaphoreType.DMA,
            pltpu.SemaphoreType.DMA,
            pltpu.SemaphoreType.DMA,
        ],
    )
    def _sc_agg(x_hbm, e_hbm, src_hbm, dst_hbm, out_hbm,
                isrc_v, idst_v, xg0, xg1, xg2, ev0, ev1, mb0, mb1, acc_sh,
                lsem0, lsem1, lsem2, esem0, esem1, ssem0, ssem1, isem):
        cid = lax.axis_index("c")
        sid = lax.axis_index("s")
        wid = cid * NS + sid
        xg = (xg0, xg1, xg2)
        ev = (ev0, ev1)
        mb = (mb0, mb1)
        lsem = (lsem0, lsem1, lsem2)
        esem = (esem0, esem1)
        ssem = (ssem0, ssem1)

        def zero_row(i, carry):
            for j in range(D // 16):
                xg0[i, pl.ds(j * 16, 16)] = jnp.zeros((16,), jnp.float32)
            return carry

        lax.fori_loop(0, C, zero_row, 0)
        for t in range(ZT):
            rchunk = sid + NS * t

            @pl.when(rchunk < NRCH)
            def _():
                pltpu.sync_copy(xg0, acc_sh.at[pl.ds(rchunk * C, C)])
        plsc.subcore_barrier()

        def islot(k):
            return lax.rem(k, NIB)

        def issue_idx(k):
            base = wid * EPW + k * C
            pltpu.async_copy(src_hbm.at[pl.ds(base, C)],
                             isrc_v.at[islot(k)], isem)
            pltpu.async_copy(dst_hbm.at[pl.ds(base, C)],
                             idst_v.at[islot(k)], isem)

        def wait_idx(k):
            base = wid * EPW + k * C
            pltpu.make_async_copy(src_hbm.at[pl.ds(base, C)],
                                  isrc_v.at[islot(k)], isem).wait()
            pltpu.make_async_copy(dst_hbm.at[pl.ds(base, C)],
                                  idst_v.at[islot(k)], isem).wait()

        def issue_g(k, b3):
            pltpu.async_copy(x_hbm.at[isrc_v.at[islot(k)]], xg[b3], lsem[b3])

        def wait_g(k, b3):
            pltpu.make_async_copy(x_hbm.at[isrc_v.at[islot(k)]], xg[b3],
                                  lsem[b3]).wait()

        def issue_e(k, b2):
            pltpu.async_copy(e_hbm.at[pl.ds(wid * EPW + k * C, C)],
                             ev[b2], esem[b2])

        def wait_e(k, b2):
            pltpu.make_async_copy(e_hbm.at[pl.ds(wid * EPW + k * C, C)],
                                  ev[b2], esem[b2]).wait()

        def compute(b3, b2):
            def row_body(i, rcarry):
                for j in range(D // 16):
                    sl = pl.ds(j * 16, 16)
                    mb[b2][i, sl] = jnp.maximum(
                        xg[b3][i, sl] + ev[b2][i, sl], 0.0)
                return rcarry

            lax.fori_loop(0, C, row_body, 0)

        def issue_scatter(k, b2):
            pltpu.async_copy(mb[b2], acc_sh.at[idst_v.at[islot(k)]],
                             ssem[b2], add=True)

        def wait_scatter(k, b2):
            pltpu.make_async_copy(mb[b2], acc_sh.at[idst_v.at[islot(k)]],
                                  ssem[b2]).wait()

        def stage(k, b3, b2, first):
            wait_g(k, b3)
            wait_e(k, b2)
            if not first:
                wait_scatter(k - 2, b2)
            compute(b3, b2)
            issue_scatter(k, b2)

            @pl.when(k + 2 < NCHUNK)
            def _():
                issue_e(k + 2, b2)

            @pl.when(k + 3 < NCHUNK)
            def _():
                wait_idx(k + 3)
                issue_g(k + 3, b3)

            @pl.when(k + 4 < NCHUNK)
            def _():
                issue_idx(k + 4)

        issue_e(0, 0)
        issue_e(1, 1)
        issue_idx(0)
        wait_idx(0)
        issue_g(0, 0)
        issue_idx(1)
        wait_idx(1)
        issue_g(1, 1)
        issue_idx(2)
        wait_idx(2)
        issue_g(2, 2)
        issue_idx(3)

        stage(0, 0, 0, True)
        stage(1, 1, 1, True)
        stage(2, 2, 0, False)
        stage(3, 0, 1, False)

        def six_body(g, carry):
            s = 4 + 6 * g
            stage(s, 1, 0, False)
            stage(s + 1, 2, 1, False)
            stage(s + 2, 0, 0, False)
            stage(s + 3, 1, 1, False)
            stage(s + 4, 2, 0, False)
            stage(s + 5, 0, 1, False)
            return carry

        lax.fori_loop(0, (NCHUNK - 4) // 6, six_body, 0)
        wait_scatter(NCHUNK - 2, 0)
        wait_scatter(NCHUNK - 1, 1)
        plsc.subcore_barrier()

        for t in range(ZT):
            rchunk = sid + NS * t

            @pl.when(rchunk < NRCH)
            def _():
                pltpu.sync_copy(acc_sh.at[pl.ds(rchunk * C, C)],
                                out_hbm.at[cid, pl.ds(rchunk * C, C)])

    return _sc_agg


@functools.cache
def _get_sc_agg_pk():
    mesh = plsc.VectorSubcoreMesh(
        core_axis_name="c", subcore_axis_name="s", num_cores=NC, num_subcores=NS)

    @functools.partial(
        pl.kernel,
        out_type=jax.ShapeDtypeStruct((NC, N, D), jnp.float32),
        mesh=mesh,
        scratch_types=[
            pltpu.VMEM((NIB, C), jnp.int32),
            pltpu.VMEM((NIB, C), jnp.int32),
            pltpu.VMEM((C, D), jnp.float32),
            pltpu.VMEM((C, D), jnp.float32),
            pltpu.VMEM((C, D), jnp.float32),
            pltpu.VMEM((C, D // 2), jnp.int32),
            pltpu.VMEM((C, D // 2), jnp.int32),
            pltpu.VMEM((C, D // 2), jnp.int32),
            pltpu.VMEM((C, D), jnp.float32),
            pltpu.VMEM((C, D), jnp.float32),
            pltpu.VMEM_SHARED((N, D), jnp.float32),
            pltpu.SemaphoreType.DMA,
            pltpu.SemaphoreType.DMA,
            pltpu.SemaphoreType.DMA,
            pltpu.SemaphoreType.DMA,
            pltpu.SemaphoreType.DMA,
            pltpu.SemaphoreType.DMA,
        ],
    )
    def _sc_agg_pk(x_hbm, e_hbm, src_hbm, dst_hbm, out_hbm,
                   isrc_v, idst_v, xg0, xg1, xg2, ev0, ev1, ev2, mb0, mb1,
                   acc_sh, lsem0, lsem1, lsem2, ssem0, ssem1, isem):
        cid = lax.axis_index("c")
        sid = lax.axis_index("s")
        wid = cid * NS + sid
        xg = (xg0, xg1, xg2)
        ev = (ev0, ev1, ev2)
        mb = (mb0, mb1)
        lsem = (lsem0, lsem1, lsem2)
        ssem = (ssem0, ssem1)

        def zero_row(i, carry):
            for j in range(D // 16):
                xg0[i, pl.ds(j * 16, 16)] = jnp.zeros((16,), jnp.float32)
            return carry

        lax.fori_loop(0, C, zero_row, 0)
        for t in range(ZT):
            rchunk = sid + NS * t

            @pl.when(rchunk < NRCH)
            def _():
                pltpu.sync_copy(xg0, acc_sh.at[pl.ds(rchunk * C, C)])
        plsc.subcore_barrier()

        def islot(k):
            return lax.rem(k, NIB)

        def issue_idx(k):
            base = wid * EPW + k * C
            pltpu.async_copy(src_hbm.at[pl.ds(base, C)],
                             isrc_v.at[islot(k)], isem)
            pltpu.async_copy(dst_hbm.at[pl.ds(base, C)],
                             idst_v.at[islot(k)], isem)

        def wait_idx(k):
            base = wid * EPW + k * C
            pltpu.make_async_copy(src_hbm.at[pl.ds(base, C)],
                                  isrc_v.at[islot(k)], isem).wait()
            pltpu.make_async_copy(dst_hbm.at[pl.ds(base, C)],
                                  idst_v.at[islot(k)], isem).wait()

        def issue_load(k, b):
            pltpu.async_copy(x_hbm.at[isrc_v.at[islot(k)]], xg[b], lsem[b])
            pltpu.async_copy(e_hbm.at[pl.ds(wid * EPW + k * C, C)],
                             ev[b], lsem[b])

        def wait_load(k, b):
            pltpu.make_async_copy(x_hbm.at[isrc_v.at[islot(k)]], xg[b],
                                  lsem[b]).wait()
            pltpu.make_async_copy(e_hbm.at[pl.ds(wid * EPW + k * C, C)],
                                  ev[b], lsem[b]).wait()

        def compute(b3, b2):
            def row_body(r, rcarry):
                for j in range(D // 32):
                    sl = pl.ds(j * 16, 16)
                    sh = pl.ds(j * 16 + D // 2, 16)
                    w = ev[b3][r, sl]
                    lo = jax.lax.bitcast_convert_type(w << 16, jnp.float32)
                    hi = jax.lax.bitcast_convert_type(
                        w & jnp.int32(-65536), jnp.float32)
                    mb[b2][r, sl] = jnp.maximum(xg[b3][r, sl] + lo, 0.0)
                    mb[b2][r, sh] = jnp.maximum(xg[b3][r, sh] + hi, 0.0)
                return rcarry

            lax.fori_loop(0, C, row_body, 0)

        def issue_scatter(k, b2):
            pltpu.async_copy(mb[b2], acc_sh.at[idst_v.at[islot(k)]],
                             ssem[b2], add=True)

        def wait_scatter(k, b2):
            pltpu.make_async_copy(mb[b2], acc_sh.at[idst_v.at[islot(k)]],
                                  ssem[b2]).wait()

        def stage(k, b3, b2, first):
            wait_load(k, b3)
            if not first:
                wait_scatter(k - 2, b2)
            compute(b3, b2)
            issue_scatter(k, b2)

            @pl.when(k + 3 < NCHUNK)
            def _():
                wait_idx(k + 3)
                issue_load(k + 3, b3)

            @pl.when(k + 4 < NCHUNK)
            def _():
                issue_idx(k + 4)

        issue_idx(0)
        wait_idx(0)
        issue_load(0, 0)
        issue_idx(1)
        wait_idx(1)
        issue_load(1, 1)
        issue_idx(2)
        wait_idx(2)
        issue_load(2, 2)
        issue_idx(3)

        stage(0, 0, 0, True)
        stage(1, 1, 1, True)
        stage(2, 2, 0, False)
        stage(3, 0, 1, False)

        def six_body(g, carry):
            s = 4 + 6 * g
            stage(s, 1, 0, False)
            stage(s + 1, 2, 1, False)
            stage(s + 2, 0, 0, False)
            stage(s + 3, 1, 1, False)
            stage(s + 4, 2, 0, False)
            stage(s + 5, 0, 1, False)
            return carry

        lax.fori_loop(0, (NCHUNK - 4) // 6, six_body, 0)
        wait_scatter(NCHUNK - 2, 0)
        wait_scatter(NCHUNK - 1, 1)
        plsc.subcore_barrier()

        for t in range(ZT):
            rchunk = sid + NS * t

            @pl.when(rchunk < NRCH)
            def _():
                pltpu.sync_copy(acc_sh.at[pl.ds(rchunk * C, C)],
                                out_hbm.at[cid, pl.ds(rchunk * C, C)])

    return _sc_agg_pk


_EBLK = 2000


def _pack_cols(y):
    yb = jax.lax.bitcast_convert_type(y.astype(jnp.bfloat16), jnp.uint16)
    yb = yb.astype(jnp.uint32)
    packed = yb[:, :D // 2] | (yb[:, D // 2:] << 16)
    return jax.lax.bitcast_convert_type(packed, jnp.int32)


def _edge_mlp_body(e_ref, w0_ref, b0_ref, w1_ref, b1_ref, y1_ref, y2_ref):
    y1 = jnp.maximum(
        jax.lax.dot_general(e_ref[...], w0_ref[...], (((1,), (0,)), ((), ())),
                            preferred_element_type=jnp.float32) + b0_ref[...], 0.0)
    y1_ref[...] = _pack_cols(y1)
    y2_ref[...] = _pack_cols(jnp.maximum(
        jax.lax.dot_general(y1, w1_ref[...], (((1,), (0,)), ((), ())),
                            preferred_element_type=jnp.float32) + b1_ref[...], 0.0))


def _edge_mlp(e, w0t, b0, w1t, b1):
    return pl.pallas_call(
        _edge_mlp_body,
        grid=(E // _EBLK,),
        in_specs=[
            pl.BlockSpec((_EBLK, D), lambda i: (i, 0)),
            pl.BlockSpec((D, D), lambda i: (0, 0)),
            pl.BlockSpec((1, D), lambda i: (0, 0)),
            pl.BlockSpec((D, D), lambda i: (0, 0)),
            pl.BlockSpec((1, D), lambda i: (0, 0)),
        ],
        out_specs=[
            pl.BlockSpec((_EBLK, D // 2), lambda i: (i, 0)),
            pl.BlockSpec((_EBLK, D // 2), lambda i: (i, 0)),
        ],
        out_shape=[
            jax.ShapeDtypeStruct((E, D // 2), jnp.int32),
            jax.ShapeDtypeStruct((E, D // 2), jnp.int32),
        ],
    )(e, w0t, b0.reshape(1, D), w1t, b1.reshape(1, D))


def _node_update_body(x_ref, p_ref, w_ref, b_ref, g_ref, be_ref, o_ref):
    h = x_ref[...] + p_ref[0] + p_ref[1]
    y = jnp.maximum(
        jax.lax.dot_general(h, w_ref[...], (((1,), (0,)), ((), ())),
                            preferred_element_type=jnp.float32) + b_ref[...], 0.0)
    mean = jnp.mean(y, axis=0, keepdims=True)
    var = jnp.mean((y - mean) ** 2, axis=0, keepdims=True)
    o_ref[...] = (y - mean) * lax.rsqrt(var + BN_EPS) * g_ref[...] + be_ref[...]


def _node_update(x, p, wt, b, g, be):
    return pl.pallas_call(
        _node_update_body,
        out_shape=jax.ShapeDtypeStruct((N, D), jnp.float32),
    )(x, p, wt, b.reshape(1, D), g.reshape(1, D), be.reshape(1, D))


def _node_final_body(x_ref, p_ref, w_ref, b_ref, init_ref, o_ref):
    h = x_ref[...] + p_ref[0] + p_ref[1]
    y = jnp.maximum(
        jax.lax.dot_general(h, w_ref[...], (((1,), (0,)), ((), ())),
                            preferred_element_type=jnp.float32) + b_ref[...], 0.0)
    o_ref[...] = y + init_ref[...]


def _node_final(x, p, wt, b, init):
    return pl.pallas_call(
        _node_final_body,
        out_shape=jax.ShapeDtypeStruct((N, D), jnp.float32),
    )(x, p, wt, b.reshape(1, D), init)


def kernel(node_feat, edge_feat, We_w, We_b, Wa_w, Wa_b, gamma, beta, edge_index):
    src = edge_index[0]
    dst = edge_index[1]

    e1p, e2p = _edge_mlp(edge_feat, We_w[0].T, We_b[0], We_w[1].T, We_b[1])

    p0 = _get_sc_agg()(node_feat, edge_feat, src, dst)
    x1 = _node_update(node_feat, p0, Wa_w[0].T, Wa_b[0], gamma[0], beta[0])
    sc_agg_pk = _get_sc_agg_pk()
    p1 = sc_agg_pk(x1, e1p, src, dst)
    x2 = _node_update(x1, p1, Wa_w[1].T, Wa_b[1], gamma[1], beta[1])
    p2 = sc_agg_pk(x2, e2p, src, dst)
    return _node_final(x2, p2, Wa_w[1].T, Wa_b[1], node_feat)
